# Initial kernel scaffold; baseline (speedup 1.0000x reference)
import jax
import jax.numpy as jnp
from jax.experimental import pallas as pl


def kernel(x_user, x_group, edge_bt, edge_inc, W1_bt, b1_bt, W1_inc, b1_inc, W2_bt, b2_bt, W2_inc, b2_inc):
    raise NotImplementedError("write your pallas kernel here")


# speedup vs baseline: 2.9488x; 2.9488x over previous
import functools

import jax
import jax.numpy as jnp
from jax import lax
from jax.experimental import pallas as pl
from jax.experimental.pallas import tpu as pltpu
from jax.experimental.pallas import tpu_sc as plsc

N = 10000
E = 320000
F = 128
NC, NS = 2, 16
W_TILES = NC * NS
CH = 128
PER_TILE = 79
E_PAD = W_TILES * PER_TILE * CH
N_ROWS = E_PAD // CH
R = 10240
STRIPE = R // NS
DUMMY = N
BLK = 256


def _mesh():
    return plsc.VectorSubcoreMesh(core_axis_name="c", subcore_axis_name="s")


def _deg_counts(sb, db, si, di):
    @functools.partial(
        pl.kernel,
        mesh=_mesh(),
        out_type=jax.ShapeDtypeStruct((NC, 4, R), jnp.float32),
        scratch_types=[
            pltpu.VMEM((CH,), jnp.int32),
            pltpu.VMEM((CH,), jnp.float32),
            pltpu.VMEM((STRIPE,), jnp.float32),
            pltpu.VMEM_SHARED((R,), jnp.float32),
            pltpu.VMEM_SHARED((R,), jnp.float32),
            pltpu.VMEM_SHARED((R,), jnp.float32),
            pltpu.VMEM_SHARED((R,), jnp.float32),
        ],
    )
    def k(sb_h, db_h, si_h, di_h, out_h, idx_v, ones_v, z_v, d0, d1, d2, d3):
        c = lax.axis_index("c")
        s = lax.axis_index("s")
        w = s * NC + c
        one16 = jnp.ones((16,), jnp.float32)
        zero16 = jnp.zeros((16,), jnp.float32)
        for j in range(CH // 16):
            ones_v[pl.ds(j * 16, 16)] = one16
        for j in range(STRIPE // 16):
            z_v[pl.ds(j * 16, 16)] = zero16
        for d in (d0, d1, d2, d3):
            pltpu.sync_copy(z_v, d.at[pl.ds(s * STRIPE, STRIPE)])
        plsc.subcore_barrier()

        def body(i, _):
            row = w * PER_TILE + i
            for idx_h, d in ((sb_h, d0), (db_h, d1), (si_h, d2), (di_h, d3)):
                pltpu.sync_copy(idx_h.at[row], idx_v)
                pltpu.sync_copy(ones_v, d.at[idx_v], add=True)
            return 0

        lax.fori_loop(0, PER_TILE, body, 0)
        plsc.subcore_barrier()
        for t, d in enumerate((d0, d1, d2, d3)):
            pltpu.sync_copy(
                d.at[pl.ds(s * STRIPE, STRIPE)],
                out_h.at[c, t, pl.ds(s * STRIPE, STRIPE)],
            )

    return k(sb, db, si, di)


def _edge_scatter(m_bt, sb, db, m_inc, si, di):
    @functools.partial(
        pl.kernel,
        mesh=_mesh(),
        out_type=(
            jax.ShapeDtypeStruct((NC, R, F), jnp.float32),
            jax.ShapeDtypeStruct((NC, R, F), jnp.float32),
        ),
        scratch_types=[
            pltpu.VMEM((CH,), jnp.int32),
            pltpu.VMEM((CH,), jnp.int32),
            pltpu.VMEM((CH, F), jnp.float32),
            pltpu.VMEM_SHARED((R, F), jnp.float32),
            pltpu.SemaphoreType.DMA,
        ],
    )
    def k(mb_h, sb_h, db_h, mi_h, si_h, di_h, ob_h, oi_h,
          is_v, id_v, rows_v, agg_sh, sem):
        c = lax.axis_index("c")
        s = lax.axis_index("s")
        w = s * NC + c
        zero16 = jnp.zeros((16,), jnp.float32)

        for m_h, s_h, d_h, o_h in ((mb_h, sb_h, db_h, ob_h),
                                   (mi_h, si_h, di_h, oi_h)):
            def zrow(i, _):
                for j in range(F // 16):
                    rows_v[i, pl.ds(j * 16, 16)] = zero16
                return 0

            lax.fori_loop(0, CH, zrow, 0)
            for kk in range(STRIPE // CH):
                pltpu.sync_copy(
                    rows_v, agg_sh.at[pl.ds(s * STRIPE + kk * CH, CH)])
            plsc.subcore_barrier()

            def body(i, _):
                row = w * PER_TILE + i
                pltpu.sync_copy(s_h.at[row], is_v)
                pltpu.sync_copy(d_h.at[row], id_v)
                pltpu.async_copy(m_h.at[is_v], rows_v, sem).wait()
                pltpu.sync_copy(rows_v, agg_sh.at[id_v], add=True)
                return 0

            lax.fori_loop(0, PER_TILE, body, 0)
            plsc.subcore_barrier()
            for kk in range(STRIPE // CH):
                r0 = s * STRIPE + kk * CH
                pltpu.sync_copy(agg_sh.at[pl.ds(r0, CH)],
                                o_h.at[c, pl.ds(r0, CH)])

    return k(m_bt, sb, db, m_inc, si, di)


def _scale(dp, t):
    return lax.rsqrt(jnp.maximum(dp[0, t] + dp[1, t], 1.0))


def _tc_layer1(xu, xg, w_bt, w_inc, deg_parts):
    def body(dp_ref, xu_ref, xg_ref, wbt_ref, winc_ref, mbt_ref, minc_ref):
        dp = dp_ref[...]
        s_bt = _scale(dp, 0)
        s_inc = _scale(dp, 2)
        mbt_ref[...] = jnp.dot(xu_ref[...] * s_bt[:, None], wbt_ref[...],
                               preferred_element_type=jnp.float32)
        minc_ref[...] = jnp.dot(xg_ref[...] * s_inc[:, None], winc_ref[...],
                                preferred_element_type=jnp.float32)

    return pl.pallas_call(
        body,
        grid=(R // BLK,),
        in_specs=[
            pl.BlockSpec((NC, 4, BLK), lambda i: (0, 0, i)),
            pl.BlockSpec((BLK, F), lambda i: (i, 0)),
            pl.BlockSpec((BLK, F), lambda i: (i, 0)),
            pl.BlockSpec((F, F), lambda i: (0, 0)),
            pl.BlockSpec((F, F), lambda i: (0, 0)),
        ],
        out_specs=(
            pl.BlockSpec((BLK, F), lambda i: (i, 0)),
            pl.BlockSpec((BLK, F), lambda i: (i, 0)),
        ),
        out_shape=(
            jax.ShapeDtypeStruct((R, F), jnp.float32),
            jax.ShapeDtypeStruct((R, F), jnp.float32),
        ),
    )(deg_parts, xu, xg, w_bt, w_inc)


def _tc_layer2_in(a_bt, a_inc, deg_parts, b1_bt, b1_inc, w2_bt, w2_inc):
    def body(dp_ref, abt_ref, ainc_ref, bbt_ref, binc_ref,
             wbt_ref, winc_ref, mbt_ref, minc_ref):
        dp = dp_ref[...]
        h_group = jnp.maximum(
            (abt_ref[0] + abt_ref[1]) * _scale(dp, 1)[:, None]
            + bbt_ref[...], 0.0)
        h_user = jnp.maximum(
            (ainc_ref[0] + ainc_ref[1]) * _scale(dp, 3)[:, None]
            + binc_ref[...], 0.0)
        mbt_ref[...] = jnp.dot(h_user * _scale(dp, 0)[:, None], wbt_ref[...],
                               preferred_element_type=jnp.float32)
        minc_ref[...] = jnp.dot(h_group * _scale(dp, 2)[:, None],
                                winc_ref[...],
                                preferred_element_type=jnp.float32)

    return pl.pallas_call(
        body,
        grid=(R // BLK,),
        in_specs=[
            pl.BlockSpec((NC, 4, BLK), lambda i: (0, 0, i)),
            pl.BlockSpec((NC, BLK, F), lambda i: (0, i, 0)),
            pl.BlockSpec((NC, BLK, F), lambda i: (0, i, 0)),
            pl.BlockSpec((1, F), lambda i: (0, 0)),
            pl.BlockSpec((1, F), lambda i: (0, 0)),
            pl.BlockSpec((F, F), lambda i: (0, 0)),
            pl.BlockSpec((F, F), lambda i: (0, 0)),
        ],
        out_specs=(
            pl.BlockSpec((BLK, F), lambda i: (i, 0)),
            pl.BlockSpec((BLK, F), lambda i: (i, 0)),
        ),
        out_shape=(
            jax.ShapeDtypeStruct((R, F), jnp.float32),
            jax.ShapeDtypeStruct((R, F), jnp.float32),
        ),
    )(deg_parts, a_bt, a_inc, b1_bt, b1_inc, w2_bt, w2_inc)


def _tc_final(a_bt, a_inc, deg_parts, b2_bt, b2_inc):
    def body(dp_ref, abt_ref, ainc_ref, bbt_ref, binc_ref, og_ref, ou_ref):
        dp = dp_ref[...]
        og_ref[...] = ((abt_ref[0] + abt_ref[1]) * _scale(dp, 1)[:, None]
                       + bbt_ref[...])
        ou_ref[...] = ((ainc_ref[0] + ainc_ref[1]) * _scale(dp, 3)[:, None]
                       + binc_ref[...])

    return pl.pallas_call(
        body,
        grid=(R // BLK,),
        in_specs=[
            pl.BlockSpec((NC, 4, BLK), lambda i: (0, 0, i)),
            pl.BlockSpec((NC, BLK, F), lambda i: (0, i, 0)),
            pl.BlockSpec((NC, BLK, F), lambda i: (0, i, 0)),
            pl.BlockSpec((1, F), lambda i: (0, 0)),
            pl.BlockSpec((1, F), lambda i: (0, 0)),
        ],
        out_specs=(
            pl.BlockSpec((BLK, F), lambda i: (i, 0)),
            pl.BlockSpec((BLK, F), lambda i: (i, 0)),
        ),
        out_shape=(
            jax.ShapeDtypeStruct((R, F), jnp.float32),
            jax.ShapeDtypeStruct((R, F), jnp.float32),
        ),
    )(deg_parts, a_bt, a_inc, b2_bt, b2_inc)


def _prep_idx(idx):
    pad = jnp.full((E_PAD - E,), DUMMY, jnp.int32)
    return jnp.concatenate([idx.astype(jnp.int32), pad]).reshape(N_ROWS, CH)


def kernel(x_user, x_group, edge_bt, edge_inc,
           W1_bt, b1_bt, W1_inc, b1_inc,
           W2_bt, b2_bt, W2_inc, b2_inc):
    xu = jnp.zeros((R, F), jnp.float32).at[:N].set(x_user)
    xg = jnp.zeros((R, F), jnp.float32).at[:N].set(x_group)
    sb, db = _prep_idx(edge_bt[0]), _prep_idx(edge_bt[1])
    si, di = _prep_idx(edge_inc[0]), _prep_idx(edge_inc[1])

    deg_parts = _deg_counts(sb, db, si, di)
    m1_bt, m1_inc = _tc_layer1(xu, xg, W1_bt, W1_inc, deg_parts)
    a1_bt, a1_inc = _edge_scatter(m1_bt, sb, db, m1_inc, si, di)
    m2_bt, m2_inc = _tc_layer2_in(
        a1_bt, a1_inc, deg_parts,
        b1_bt.reshape(1, F), b1_inc.reshape(1, F), W2_bt, W2_inc)
    a2_bt, a2_inc = _edge_scatter(m2_bt, sb, db, m2_inc, si, di)
    out_group, out_user = _tc_final(
        a2_bt, a2_inc, deg_parts,
        b2_bt.reshape(1, F), b2_inc.reshape(1, F))
    return out_user[:N], out_group[:N]


# speedup vs baseline: 9.1619x; 3.1070x over previous
import functools

import jax
import jax.numpy as jnp
from jax import lax
from jax.experimental import pallas as pl
from jax.experimental.pallas import tpu as pltpu
from jax.experimental.pallas import tpu_sc as plsc

N = 10000
E = 320000
F = 128
NC, NS = 2, 16
W_TILES = NC * NS
CH = 128
PER_TILE = 80
HALF = PER_TILE // 2
E_PAD = W_TILES * PER_TILE * CH
N_ROWS = E_PAD // CH
R = 10240
STRIPE = R // NS
N_DUMMY = R - N
BLK = 256


def _mesh():
    return plsc.VectorSubcoreMesh(core_axis_name="c", subcore_axis_name="s")


def _deg_counts(sb, db, si, di):
    @functools.partial(
        pl.kernel,
        mesh=_mesh(),
        out_type=jax.ShapeDtypeStruct((NC, 4, R), jnp.float32),
        scratch_types=[
            pltpu.VMEM((PER_TILE, CH), jnp.int32),
            pltpu.VMEM((PER_TILE, CH), jnp.int32),
            pltpu.VMEM((PER_TILE, CH), jnp.int32),
            pltpu.VMEM((PER_TILE, CH), jnp.int32),
            pltpu.VMEM((CH,), jnp.float32),
            pltpu.VMEM((STRIPE,), jnp.float32),
            pltpu.VMEM_SHARED((R,), jnp.float32),
            pltpu.VMEM_SHARED((R,), jnp.float32),
            pltpu.VMEM_SHARED((R,), jnp.float32),
            pltpu.VMEM_SHARED((R,), jnp.float32),
            pltpu.SemaphoreType.DMA,
        ],
    )
    def k(sb_h, db_h, si_h, di_h, out_h,
          i0_v, i1_v, i2_v, i3_v, ones_v, z_v, d0, d1, d2, d3, sem):
        c = lax.axis_index("c")
        s = lax.axis_index("s")
        w = s * NC + c
        one16 = jnp.ones((16,), jnp.float32)
        zero16 = jnp.zeros((16,), jnp.float32)
        for j in range(CH // 16):
            ones_v[pl.ds(j * 16, 16)] = one16
        for j in range(STRIPE // 16):
            z_v[pl.ds(j * 16, 16)] = zero16
        for idx_h, i_v in ((sb_h, i0_v), (db_h, i1_v), (si_h, i2_v),
                           (di_h, i3_v)):
            pltpu.sync_copy(idx_h.at[pl.ds(w * PER_TILE, PER_TILE)], i_v)
        for d in (d0, d1, d2, d3):
            pltpu.sync_copy(z_v, d.at[pl.ds(s * STRIPE, STRIPE)])
        plsc.subcore_barrier()

        def body(i, _):
            descs = [
                pltpu.async_copy(ones_v, d.at[i_v.at[i]], sem, add=True)
                for i_v, d in ((i0_v, d0), (i1_v, d1), (i2_v, d2), (i3_v, d3))
            ]
            for desc in descs:
                desc.wait()
            return 0

        lax.fori_loop(0, PER_TILE, body, 0)
        plsc.subcore_barrier()
        for t, d in enumerate((d0, d1, d2, d3)):
            pltpu.sync_copy(
                d.at[pl.ds(s * STRIPE, STRIPE)],
                out_h.at[c, t, pl.ds(s * STRIPE, STRIPE)],
            )

    return k(sb, db, si, di)


def _edge_scatter(m_bt, sb, db, m_inc, si, di):
    @functools.partial(
        pl.kernel,
        mesh=_mesh(),
        out_type=(
            jax.ShapeDtypeStruct((NC, R, F), jnp.float32),
            jax.ShapeDtypeStruct((NC, R, F), jnp.float32),
        ),
        scratch_types=[
            pltpu.VMEM((HALF, CH), jnp.int32),
            pltpu.VMEM((HALF, CH), jnp.int32),
            pltpu.VMEM((CH, F), jnp.float32),
            pltpu.VMEM((CH, F), jnp.float32),
            pltpu.VMEM_SHARED((R, F), jnp.float32),
            pltpu.SemaphoreType.DMA,
            pltpu.SemaphoreType.DMA,
        ],
    )
    def k(mb_h, sb_h, db_h, mi_h, si_h, di_h, ob_h, oi_h,
          is_v, id_v, rows0_v, rows1_v, agg_sh, sem0, sem1):
        c = lax.axis_index("c")
        s = lax.axis_index("s")
        w = s * NC + c
        zero16 = jnp.zeros((16,), jnp.float32)
        bufs = (rows0_v, rows1_v)
        sems = (sem0, sem1)

        for m_h, s_h, d_h, o_h in ((mb_h, sb_h, db_h, ob_h),
                                   (mi_h, si_h, di_h, oi_h)):
            def zrow(i, _):
                for j in range(F // 16):
                    rows0_v[i, pl.ds(j * 16, 16)] = zero16
                return 0

            lax.fori_loop(0, CH, zrow, 0)
            for kk in range(STRIPE // CH):
                pltpu.sync_copy(
                    rows0_v, agg_sh.at[pl.ds(s * STRIPE + kk * CH, CH)])
            plsc.subcore_barrier()

            for ph in range(2):
                base = w * PER_TILE + ph * HALF
                pltpu.sync_copy(s_h.at[pl.ds(base, HALF)], is_v)
                pltpu.sync_copy(d_h.at[pl.ds(base, HALF)], id_v)
                pltpu.async_copy(m_h.at[is_v.at[0]], rows0_v, sem0)

                def body(j, _):
                    for b in range(2):
                        i = 2 * j + b
                        nb = 1 - b

                        @pl.when(i + 1 < HALF)
                        def _():
                            pltpu.async_copy(
                                m_h.at[is_v.at[i + 1]], bufs[nb], sems[nb])

                        pltpu.make_async_copy(
                            m_h.at[is_v.at[i]], bufs[b], sems[b]).wait()
                        pltpu.sync_copy(
                            bufs[b], agg_sh.at[id_v.at[i]], add=True)
                    return 0

                lax.fori_loop(0, HALF // 2, body, 0)
            plsc.subcore_barrier()
            for kk in range(STRIPE // CH):
                r0 = s * STRIPE + kk * CH
                pltpu.sync_copy(agg_sh.at[pl.ds(r0, CH)],
                                o_h.at[c, pl.ds(r0, CH)])

    return k(m_bt, sb, db, m_inc, si, di)


def _scale(dp, t):
    return lax.rsqrt(jnp.maximum(dp[0, t] + dp[1, t], 1.0))


def _tc_layer1(xu, xg, w_bt, w_inc, deg_parts):
    def body(dp_ref, xu_ref, xg_ref, wbt_ref, winc_ref, mbt_ref, minc_ref):
        dp = dp_ref[...]
        s_bt = _scale(dp, 0)
        s_inc = _scale(dp, 2)
        mbt_ref[...] = jnp.dot(xu_ref[...] * s_bt[:, None], wbt_ref[...],
                               preferred_element_type=jnp.float32)
        minc_ref[...] = jnp.dot(xg_ref[...] * s_inc[:, None], winc_ref[...],
                                preferred_element_type=jnp.float32)

    return pl.pallas_call(
        body,
        grid=(R // BLK,),
        in_specs=[
            pl.BlockSpec((NC, 4, BLK), lambda i: (0, 0, i)),
            pl.BlockSpec((BLK, F), lambda i: (i, 0)),
            pl.BlockSpec((BLK, F), lambda i: (i, 0)),
            pl.BlockSpec((F, F), lambda i: (0, 0)),
            pl.BlockSpec((F, F), lambda i: (0, 0)),
        ],
        out_specs=(
            pl.BlockSpec((BLK, F), lambda i: (i, 0)),
            pl.BlockSpec((BLK, F), lambda i: (i, 0)),
        ),
        out_shape=(
            jax.ShapeDtypeStruct((R, F), jnp.float32),
            jax.ShapeDtypeStruct((R, F), jnp.float32),
        ),
    )(deg_parts, xu, xg, w_bt, w_inc)


def _tc_layer2_in(a_bt, a_inc, deg_parts, b1_bt, b1_inc, w2_bt, w2_inc):
    def body(dp_ref, abt_ref, ainc_ref, bbt_ref, binc_ref,
             wbt_ref, winc_ref, mbt_ref, minc_ref):
        dp = dp_ref[...]
        h_group = jnp.maximum(
            (abt_ref[0] + abt_ref[1]) * _scale(dp, 1)[:, None]
            + bbt_ref[...], 0.0)
        h_user = jnp.maximum(
            (ainc_ref[0] + ainc_ref[1]) * _scale(dp, 3)[:, None]
            + binc_ref[...], 0.0)
        mbt_ref[...] = jnp.dot(h_user * _scale(dp, 0)[:, None], wbt_ref[...],
                               preferred_element_type=jnp.float32)
        minc_ref[...] = jnp.dot(h_group * _scale(dp, 2)[:, None],
                                winc_ref[...],
                                preferred_element_type=jnp.float32)

    return pl.pallas_call(
        body,
        grid=(R // BLK,),
        in_specs=[
            pl.BlockSpec((NC, 4, BLK), lambda i: (0, 0, i)),
            pl.BlockSpec((NC, BLK, F), lambda i: (0, i, 0)),
            pl.BlockSpec((NC, BLK, F), lambda i: (0, i, 0)),
            pl.BlockSpec((1, F), lambda i: (0, 0)),
            pl.BlockSpec((1, F), lambda i: (0, 0)),
            pl.BlockSpec((F, F), lambda i: (0, 0)),
            pl.BlockSpec((F, F), lambda i: (0, 0)),
        ],
        out_specs=(
            pl.BlockSpec((BLK, F), lambda i: (i, 0)),
            pl.BlockSpec((BLK, F), lambda i: (i, 0)),
        ),
        out_shape=(
            jax.ShapeDtypeStruct((R, F), jnp.float32),
            jax.ShapeDtypeStruct((R, F), jnp.float32),
        ),
    )(deg_parts, a_bt, a_inc, b1_bt, b1_inc, w2_bt, w2_inc)


def _tc_final(a_bt, a_inc, deg_parts, b2_bt, b2_inc):
    def body(dp_ref, abt_ref, ainc_ref, bbt_ref, binc_ref, og_ref, ou_ref):
        dp = dp_ref[...]
        og_ref[...] = ((abt_ref[0] + abt_ref[1]) * _scale(dp, 1)[:, None]
                       + bbt_ref[...])
        ou_ref[...] = ((ainc_ref[0] + ainc_ref[1]) * _scale(dp, 3)[:, None]
                       + binc_ref[...])

    return pl.pallas_call(
        body,
        grid=(R // BLK,),
        in_specs=[
            pl.BlockSpec((NC, 4, BLK), lambda i: (0, 0, i)),
            pl.BlockSpec((NC, BLK, F), lambda i: (0, i, 0)),
            pl.BlockSpec((NC, BLK, F), lambda i: (0, i, 0)),
            pl.BlockSpec((1, F), lambda i: (0, 0)),
            pl.BlockSpec((1, F), lambda i: (0, 0)),
        ],
        out_specs=(
            pl.BlockSpec((BLK, F), lambda i: (i, 0)),
            pl.BlockSpec((BLK, F), lambda i: (i, 0)),
        ),
        out_shape=(
            jax.ShapeDtypeStruct((R, F), jnp.float32),
            jax.ShapeDtypeStruct((R, F), jnp.float32),
        ),
    )(deg_parts, a_bt, a_inc, b2_bt, b2_inc)


def _prep_idx(idx):
    pad = N + (jnp.arange(E_PAD - E, dtype=jnp.int32) % N_DUMMY)
    return jnp.concatenate([idx.astype(jnp.int32), pad]).reshape(N_ROWS, CH)


def kernel(x_user, x_group, edge_bt, edge_inc,
           W1_bt, b1_bt, W1_inc, b1_inc,
           W2_bt, b2_bt, W2_inc, b2_inc):
    xu = jnp.zeros((R, F), jnp.float32).at[:N].set(x_user)
    xg = jnp.zeros((R, F), jnp.float32).at[:N].set(x_group)
    sb, db = _prep_idx(edge_bt[0]), _prep_idx(edge_bt[1])
    si, di = _prep_idx(edge_inc[0]), _prep_idx(edge_inc[1])

    deg_parts = _deg_counts(sb, db, si, di)
    m1_bt, m1_inc = _tc_layer1(xu, xg, W1_bt, W1_inc, deg_parts)
    a1_bt, a1_inc = _edge_scatter(m1_bt, sb, db, m1_inc, si, di)
    m2_bt, m2_inc = _tc_layer2_in(
        a1_bt, a1_inc, deg_parts,
        b1_bt.reshape(1, F), b1_inc.reshape(1, F), W2_bt, W2_inc)
    a2_bt, a2_inc = _edge_scatter(m2_bt, sb, db, m2_inc, si, di)
    out_group, out_user = _tc_final(
        a2_bt, a2_inc, deg_parts,
        b2_bt.reshape(1, F), b2_inc.reshape(1, F))
    return out_user[:N], out_group[:N]


# speedup vs baseline: 9.3291x; 1.0183x over previous
import functools

import jax
import jax.numpy as jnp
from jax import lax
from jax.experimental import pallas as pl
from jax.experimental.pallas import tpu as pltpu
from jax.experimental.pallas import tpu_sc as plsc

N = 10000
E = 320000
F = 128
NC, NS = 2, 16
W_TILES = NC * NS
CH = 128
PER_TILE = 80
HALF = PER_TILE // 2
E_PAD = W_TILES * PER_TILE * CH
N_ROWS = E_PAD // CH
R = 10240
STRIPE = R // NS
N_DUMMY = R - N
BLK = 256


def _mesh():
    return plsc.VectorSubcoreMesh(core_axis_name="c", subcore_axis_name="s")


def _deg_counts(sb, db, si, di):
    @functools.partial(
        pl.kernel,
        mesh=_mesh(),
        out_type=jax.ShapeDtypeStruct((NC, 4, R), jnp.float32),
        scratch_types=[
            pltpu.VMEM((PER_TILE, CH), jnp.int32),
            pltpu.VMEM((PER_TILE, CH), jnp.int32),
            pltpu.VMEM((PER_TILE, CH), jnp.int32),
            pltpu.VMEM((PER_TILE, CH), jnp.int32),
            pltpu.VMEM((CH,), jnp.float32),
            pltpu.VMEM((STRIPE,), jnp.float32),
            pltpu.VMEM_SHARED((R,), jnp.float32),
            pltpu.VMEM_SHARED((R,), jnp.float32),
            pltpu.VMEM_SHARED((R,), jnp.float32),
            pltpu.VMEM_SHARED((R,), jnp.float32),
            pltpu.SemaphoreType.DMA,
        ],
    )
    def k(sb_h, db_h, si_h, di_h, out_h,
          i0_v, i1_v, i2_v, i3_v, ones_v, z_v, d0, d1, d2, d3, sem):
        c = lax.axis_index("c")
        s = lax.axis_index("s")
        w = s * NC + c
        one16 = jnp.ones((16,), jnp.float32)
        zero16 = jnp.zeros((16,), jnp.float32)
        for j in range(CH // 16):
            ones_v[pl.ds(j * 16, 16)] = one16
        for j in range(STRIPE // 16):
            z_v[pl.ds(j * 16, 16)] = zero16
        for idx_h, i_v in ((sb_h, i0_v), (db_h, i1_v), (si_h, i2_v),
                           (di_h, i3_v)):
            pltpu.sync_copy(idx_h.at[pl.ds(w * PER_TILE, PER_TILE)], i_v)
        for d in (d0, d1, d2, d3):
            pltpu.sync_copy(z_v, d.at[pl.ds(s * STRIPE, STRIPE)])
        plsc.subcore_barrier()

        def body(i, _):
            descs = [
                pltpu.async_copy(ones_v, d.at[i_v.at[i]], sem, add=True)
                for i_v, d in ((i0_v, d0), (i1_v, d1), (i2_v, d2), (i3_v, d3))
            ]
            for desc in descs:
                desc.wait()
            return 0

        lax.fori_loop(0, PER_TILE, body, 0)
        plsc.subcore_barrier()
        for t, d in enumerate((d0, d1, d2, d3)):
            pltpu.sync_copy(
                d.at[pl.ds(s * STRIPE, STRIPE)],
                out_h.at[c, t, pl.ds(s * STRIPE, STRIPE)],
            )

    return k(sb, db, si, di)


def _edge_scatter(m_bt, sb, db, m_inc, si, di):
    @functools.partial(
        pl.kernel,
        mesh=_mesh(),
        out_type=(
            jax.ShapeDtypeStruct((NC, R, F), jnp.float32),
            jax.ShapeDtypeStruct((NC, R, F), jnp.float32),
        ),
        scratch_types=[
            pltpu.VMEM((HALF, CH), jnp.int32),
            pltpu.VMEM((HALF, CH), jnp.int32),
            pltpu.VMEM((CH, F), jnp.float32),
            pltpu.VMEM((CH, F), jnp.float32),
            pltpu.VMEM_SHARED((R, F), jnp.float32),
            pltpu.SemaphoreType.DMA,
            pltpu.SemaphoreType.DMA,
        ],
    )
    def k(mb_h, sb_h, db_h, mi_h, si_h, di_h, ob_h, oi_h,
          is_v, id_v, rows0_v, rows1_v, agg_sh, sem0, sem1):
        c = lax.axis_index("c")
        s = lax.axis_index("s")
        w = s * NC + c
        zero16 = jnp.zeros((16,), jnp.float32)
        bufs = (rows0_v, rows1_v)
        sems = (sem0, sem1)

        for m_h, s_h, d_h, o_h in ((mb_h, sb_h, db_h, ob_h),
                                   (mi_h, si_h, di_h, oi_h)):
            def zrow(i, _):
                for j in range(F // 16):
                    rows0_v[i, pl.ds(j * 16, 16)] = zero16
                return 0

            lax.fori_loop(0, CH, zrow, 0)
            for kk in range(STRIPE // CH):
                pltpu.sync_copy(
                    rows0_v, agg_sh.at[pl.ds(s * STRIPE + kk * CH, CH)])
            plsc.subcore_barrier()

            for ph in range(2):
                base = w * PER_TILE + ph * HALF
                pltpu.sync_copy(s_h.at[pl.ds(base, HALF)], is_v)
                pltpu.sync_copy(d_h.at[pl.ds(base, HALF)], id_v)
                pltpu.async_copy(m_h.at[is_v.at[0]], rows0_v, sem0)

                def body(j, _):
                    for b in range(2):
                        i = 2 * j + b
                        nb = 1 - b

                        @pl.when(i + 1 < HALF)
                        def _():
                            pltpu.async_copy(
                                m_h.at[is_v.at[i + 1]], bufs[nb], sems[nb])

                        pltpu.make_async_copy(
                            m_h.at[is_v.at[i]], bufs[b], sems[b]).wait()
                        pltpu.sync_copy(
                            bufs[b], agg_sh.at[id_v.at[i]], add=True)
                    return 0

                lax.fori_loop(0, HALF // 2, body, 0)
            plsc.subcore_barrier()
            for kk in range(STRIPE // CH):
                r0 = s * STRIPE + kk * CH
                pltpu.sync_copy(agg_sh.at[pl.ds(r0, CH)],
                                o_h.at[c, pl.ds(r0, CH)])

    return k(m_bt, sb, db, m_inc, si, di)


def _scale(dp, t):
    return lax.rsqrt(jnp.maximum(dp[0, t] + dp[1, t], 1.0))


def _tc_layer1(xu, xg, w_bt, w_inc, deg_parts):
    def body(dp_ref, xu_ref, xg_ref, wbt_ref, winc_ref, mbt_ref, minc_ref):
        dp = dp_ref[...]
        s_bt = _scale(dp, 0)
        s_inc = _scale(dp, 2)
        mbt_ref[...] = jnp.dot(xu_ref[...] * s_bt[:, None], wbt_ref[...],
                               preferred_element_type=jnp.float32)
        minc_ref[...] = jnp.dot(xg_ref[...] * s_inc[:, None], winc_ref[...],
                                preferred_element_type=jnp.float32)

    return pl.pallas_call(
        body,
        grid=(R // BLK,),
        in_specs=[
            pl.BlockSpec((NC, 4, BLK), lambda i: (0, 0, i)),
            pl.BlockSpec((BLK, F), lambda i: (i, 0)),
            pl.BlockSpec((BLK, F), lambda i: (i, 0)),
            pl.BlockSpec((F, F), lambda i: (0, 0)),
            pl.BlockSpec((F, F), lambda i: (0, 0)),
        ],
        out_specs=(
            pl.BlockSpec((BLK, F), lambda i: (i, 0)),
            pl.BlockSpec((BLK, F), lambda i: (i, 0)),
        ),
        out_shape=(
            jax.ShapeDtypeStruct((R, F), jnp.float32),
            jax.ShapeDtypeStruct((R, F), jnp.float32),
        ),
    )(deg_parts, xu, xg, w_bt, w_inc)


def _tc_layer2_in(a_bt, a_inc, deg_parts, b1_bt, b1_inc, w2_bt, w2_inc):
    def body(dp_ref, abt_ref, ainc_ref, bbt_ref, binc_ref,
             wbt_ref, winc_ref, mbt_ref, minc_ref):
        dp = dp_ref[...]
        h_group = jnp.maximum(
            (abt_ref[0] + abt_ref[1]) * _scale(dp, 1)[:, None]
            + bbt_ref[...], 0.0)
        h_user = jnp.maximum(
            (ainc_ref[0] + ainc_ref[1]) * _scale(dp, 3)[:, None]
            + binc_ref[...], 0.0)
        mbt_ref[...] = jnp.dot(h_user * _scale(dp, 0)[:, None], wbt_ref[...],
                               preferred_element_type=jnp.float32)
        minc_ref[...] = jnp.dot(h_group * _scale(dp, 2)[:, None],
                                winc_ref[...],
                                preferred_element_type=jnp.float32)

    return pl.pallas_call(
        body,
        grid=(R // BLK,),
        in_specs=[
            pl.BlockSpec((NC, 4, BLK), lambda i: (0, 0, i)),
            pl.BlockSpec((NC, BLK, F), lambda i: (0, i, 0)),
            pl.BlockSpec((NC, BLK, F), lambda i: (0, i, 0)),
            pl.BlockSpec((1, F), lambda i: (0, 0)),
            pl.BlockSpec((1, F), lambda i: (0, 0)),
            pl.BlockSpec((F, F), lambda i: (0, 0)),
            pl.BlockSpec((F, F), lambda i: (0, 0)),
        ],
        out_specs=(
            pl.BlockSpec((BLK, F), lambda i: (i, 0)),
            pl.BlockSpec((BLK, F), lambda i: (i, 0)),
        ),
        out_shape=(
            jax.ShapeDtypeStruct((R, F), jnp.float32),
            jax.ShapeDtypeStruct((R, F), jnp.float32),
        ),
    )(deg_parts, a_bt, a_inc, b1_bt, b1_inc, w2_bt, w2_inc)


def _tc_final(a_bt, a_inc, deg_parts, b2_bt, b2_inc):
    def body(dp_ref, abt_ref, ainc_ref, bbt_ref, binc_ref, og_ref, ou_ref):
        dp = dp_ref[...]
        og_ref[...] = ((abt_ref[0] + abt_ref[1]) * _scale(dp, 1)[:, None]
                       + bbt_ref[...])
        ou_ref[...] = ((ainc_ref[0] + ainc_ref[1]) * _scale(dp, 3)[:, None]
                       + binc_ref[...])

    return pl.pallas_call(
        body,
        grid=(R // BLK,),
        in_specs=[
            pl.BlockSpec((NC, 4, BLK), lambda i: (0, 0, i)),
            pl.BlockSpec((NC, BLK, F), lambda i: (0, i, 0)),
            pl.BlockSpec((NC, BLK, F), lambda i: (0, i, 0)),
            pl.BlockSpec((1, F), lambda i: (0, 0)),
            pl.BlockSpec((1, F), lambda i: (0, 0)),
        ],
        out_specs=(
            pl.BlockSpec((BLK, F), lambda i: (i, 0)),
            pl.BlockSpec((BLK, F), lambda i: (i, 0)),
        ),
        out_shape=(
            jax.ShapeDtypeStruct((N, F), jnp.float32),
            jax.ShapeDtypeStruct((N, F), jnp.float32),
        ),
    )(deg_parts, a_bt, a_inc, b2_bt, b2_inc)


def _prep_idx(idx):
    pad = N + (jnp.arange(E_PAD - E, dtype=jnp.int32) % N_DUMMY)
    return jnp.concatenate([idx.astype(jnp.int32), pad]).reshape(N_ROWS, CH)


def kernel(x_user, x_group, edge_bt, edge_inc,
           W1_bt, b1_bt, W1_inc, b1_inc,
           W2_bt, b2_bt, W2_inc, b2_inc):
    sb, db = _prep_idx(edge_bt[0]), _prep_idx(edge_bt[1])
    si, di = _prep_idx(edge_inc[0]), _prep_idx(edge_inc[1])

    deg_parts = _deg_counts(sb, db, si, di)
    m1_bt, m1_inc = _tc_layer1(x_user, x_group, W1_bt, W1_inc, deg_parts)
    a1_bt, a1_inc = _edge_scatter(m1_bt, sb, db, m1_inc, si, di)
    m2_bt, m2_inc = _tc_layer2_in(
        a1_bt, a1_inc, deg_parts,
        b1_bt.reshape(1, F), b1_inc.reshape(1, F), W2_bt, W2_inc)
    a2_bt, a2_inc = _edge_scatter(m2_bt, sb, db, m2_inc, si, di)
    out_group, out_user = _tc_final(
        a2_bt, a2_inc, deg_parts,
        b2_bt.reshape(1, F), b2_inc.reshape(1, F))
    return out_user, out_group


# speedup vs baseline: 9.6897x; 1.0387x over previous
import functools

import jax
import jax.numpy as jnp
from jax import lax
from jax.experimental import pallas as pl
from jax.experimental.pallas import tpu as pltpu
from jax.experimental.pallas import tpu_sc as plsc

N = 10000
E = 320000
F = 128
NC, NS = 2, 16
W_TILES = NC * NS
CH = 128
PER_TILE = 80
HALF = PER_TILE // 2
E_PAD = W_TILES * PER_TILE * CH
N_ROWS = E_PAD // CH
R = 10240
STRIPE = R // NS
N_DUMMY = R - N
BLK = 256


def _mesh():
    return plsc.VectorSubcoreMesh(core_axis_name="c", subcore_axis_name="s")


def _deg_counts(sb, db, si, di):
    @functools.partial(
        pl.kernel,
        mesh=_mesh(),
        out_type=jax.ShapeDtypeStruct((NC, 4, R), jnp.float32),
        scratch_types=[
            pltpu.VMEM((PER_TILE, CH), jnp.int32),
            pltpu.VMEM((PER_TILE, CH), jnp.int32),
            pltpu.VMEM((PER_TILE, CH), jnp.int32),
            pltpu.VMEM((PER_TILE, CH), jnp.int32),
            pltpu.VMEM((CH,), jnp.float32),
            pltpu.VMEM((STRIPE,), jnp.float32),
            pltpu.VMEM_SHARED((R,), jnp.float32),
            pltpu.VMEM_SHARED((R,), jnp.float32),
            pltpu.VMEM_SHARED((R,), jnp.float32),
            pltpu.VMEM_SHARED((R,), jnp.float32),
            pltpu.SemaphoreType.DMA,
        ],
    )
    def k(sb_h, db_h, si_h, di_h, out_h,
          i0_v, i1_v, i2_v, i3_v, ones_v, z_v, d0, d1, d2, d3, sem):
        c = lax.axis_index("c")
        s = lax.axis_index("s")
        w = s * NC + c
        one16 = jnp.ones((16,), jnp.float32)
        zero16 = jnp.zeros((16,), jnp.float32)
        for j in range(CH // 16):
            ones_v[pl.ds(j * 16, 16)] = one16
        for j in range(STRIPE // 16):
            z_v[pl.ds(j * 16, 16)] = zero16
        for idx_h, i_v in ((sb_h, i0_v), (db_h, i1_v), (si_h, i2_v),
                           (di_h, i3_v)):
            pltpu.sync_copy(idx_h.at[pl.ds(w * PER_TILE, PER_TILE)], i_v)
        for d in (d0, d1, d2, d3):
            pltpu.sync_copy(z_v, d.at[pl.ds(s * STRIPE, STRIPE)])
        plsc.subcore_barrier()

        def body(i, _):
            descs = [
                pltpu.async_copy(ones_v, d.at[i_v.at[i]], sem, add=True)
                for i_v, d in ((i0_v, d0), (i1_v, d1), (i2_v, d2), (i3_v, d3))
            ]
            for desc in descs:
                desc.wait()
            return 0

        lax.fori_loop(0, PER_TILE, body, 0)
        plsc.subcore_barrier()
        for t, d in enumerate((d0, d1, d2, d3)):
            pltpu.sync_copy(
                d.at[pl.ds(s * STRIPE, STRIPE)],
                out_h.at[c, t, pl.ds(s * STRIPE, STRIPE)],
            )

    return k(sb, db, si, di)


def _edge_scatter(m, s2d, d2d):
    @functools.partial(
        pl.kernel,
        mesh=_mesh(),
        out_type=jax.ShapeDtypeStruct((NC, R, F), jnp.float32),
        scratch_types=[
            pltpu.VMEM((HALF, CH), jnp.int32),
            pltpu.VMEM((HALF, CH), jnp.int32),
            pltpu.VMEM((CH, F), jnp.float32),
            pltpu.VMEM((CH, F), jnp.float32),
            pltpu.VMEM_SHARED((R, F), jnp.float32),
            pltpu.SemaphoreType.DMA,
            pltpu.SemaphoreType.DMA,
        ],
    )
    def k(m_h, s_h, d_h, o_h, is_v, id_v, rows0_v, rows1_v, agg_sh,
          sem0, sem1):
        c = lax.axis_index("c")
        s = lax.axis_index("s")
        w = s * NC + c
        zero16 = jnp.zeros((16,), jnp.float32)
        bufs = (rows0_v, rows1_v)
        sems = (sem0, sem1)

        def zrow(i, _):
            for j in range(F // 16):
                rows0_v[i, pl.ds(j * 16, 16)] = zero16
            return 0

        lax.fori_loop(0, CH, zrow, 0)
        for kk in range(STRIPE // CH):
            pltpu.sync_copy(
                rows0_v, agg_sh.at[pl.ds(s * STRIPE + kk * CH, CH)])
        plsc.subcore_barrier()

        for ph in range(2):
            base = w * PER_TILE + ph * HALF
            pltpu.sync_copy(s_h.at[pl.ds(base, HALF)], is_v)
            pltpu.sync_copy(d_h.at[pl.ds(base, HALF)], id_v)
            pltpu.async_copy(m_h.at[is_v.at[0]], rows0_v, sem0)

            def body(j, _):
                for b in range(2):
                    i = 2 * j + b
                    nb = 1 - b

                    @pl.when(i + 1 < HALF)
                    def _():
                        pltpu.async_copy(
                            m_h.at[is_v.at[i + 1]], bufs[nb], sems[nb])

                    pltpu.make_async_copy(
                        m_h.at[is_v.at[i]], bufs[b], sems[b]).wait()
                    pltpu.sync_copy(bufs[b], agg_sh.at[id_v.at[i]], add=True)
                return 0

            lax.fori_loop(0, HALF // 2, body, 0)
        plsc.subcore_barrier()
        for kk in range(STRIPE // CH):
            r0 = s * STRIPE + kk * CH
            pltpu.sync_copy(agg_sh.at[pl.ds(r0, CH)], o_h.at[c, pl.ds(r0, CH)])

    return k(m, s2d, d2d)


def _scale(dp, t):
    return lax.rsqrt(jnp.maximum(dp[0, t] + dp[1, t], 1.0))


def _tc_m1(x, w, deg_parts, t):
    def body(dp_ref, x_ref, w_ref, m_ref):
        sc = _scale(dp_ref[...], t)
        m_ref[...] = jnp.dot(x_ref[...] * sc[:, None], w_ref[...],
                             preferred_element_type=jnp.float32)

    return pl.pallas_call(
        body,
        grid=(R // BLK,),
        in_specs=[
            pl.BlockSpec((NC, 4, BLK), lambda i: (0, 0, i)),
            pl.BlockSpec((BLK, F), lambda i: (i, 0)),
            pl.BlockSpec((F, F), lambda i: (0, 0)),
        ],
        out_specs=pl.BlockSpec((BLK, F), lambda i: (i, 0)),
        out_shape=jax.ShapeDtypeStruct((R, F), jnp.float32),
    )(deg_parts, x, w)


def _tc_h_m2(a_parts, deg_parts, t_in, b1, t_out, w2):
    def body(dp_ref, a_ref, b_ref, w_ref, m_ref):
        dp = dp_ref[...]
        h = jnp.maximum(
            (a_ref[0] + a_ref[1]) * _scale(dp, t_in)[:, None] + b_ref[...],
            0.0)
        m_ref[...] = jnp.dot(h * _scale(dp, t_out)[:, None], w_ref[...],
                             preferred_element_type=jnp.float32)

    return pl.pallas_call(
        body,
        grid=(R // BLK,),
        in_specs=[
            pl.BlockSpec((NC, 4, BLK), lambda i: (0, 0, i)),
            pl.BlockSpec((NC, BLK, F), lambda i: (0, i, 0)),
            pl.BlockSpec((1, F), lambda i: (0, 0)),
            pl.BlockSpec((F, F), lambda i: (0, 0)),
        ],
        out_specs=pl.BlockSpec((BLK, F), lambda i: (i, 0)),
        out_shape=jax.ShapeDtypeStruct((R, F), jnp.float32),
    )(deg_parts, a_parts, b1, w2)


def _tc_final(a_parts, deg_parts, t_in, b2):
    def body(dp_ref, a_ref, b_ref, o_ref):
        dp = dp_ref[...]
        o_ref[...] = ((a_ref[0] + a_ref[1]) * _scale(dp, t_in)[:, None]
                      + b_ref[...])

    return pl.pallas_call(
        body,
        grid=(R // BLK,),
        in_specs=[
            pl.BlockSpec((NC, 4, BLK), lambda i: (0, 0, i)),
            pl.BlockSpec((NC, BLK, F), lambda i: (0, i, 0)),
            pl.BlockSpec((1, F), lambda i: (0, 0)),
        ],
        out_specs=pl.BlockSpec((BLK, F), lambda i: (i, 0)),
        out_shape=jax.ShapeDtypeStruct((N, F), jnp.float32),
    )(deg_parts, a_parts, b2)


def _prep_idx(idx):
    pad = N + (jnp.arange(E_PAD - E, dtype=jnp.int32) % N_DUMMY)
    return jnp.concatenate([idx.astype(jnp.int32), pad]).reshape(N_ROWS, CH)


def kernel(x_user, x_group, edge_bt, edge_inc,
           W1_bt, b1_bt, W1_inc, b1_inc,
           W2_bt, b2_bt, W2_inc, b2_inc):
    sb, db = _prep_idx(edge_bt[0]), _prep_idx(edge_bt[1])
    si, di = _prep_idx(edge_inc[0]), _prep_idx(edge_inc[1])

    deg_parts = _deg_counts(sb, db, si, di)
    m1_bt = _tc_m1(x_user, W1_bt, deg_parts, 0)
    m1_inc = _tc_m1(x_group, W1_inc, deg_parts, 2)
    a1_bt = _edge_scatter(m1_bt, sb, db)
    a1_inc = _edge_scatter(m1_inc, si, di)
    m2_inc = _tc_h_m2(a1_bt, deg_parts, 1, b1_bt.reshape(1, F), 2, W2_inc)
    m2_bt = _tc_h_m2(a1_inc, deg_parts, 3, b1_inc.reshape(1, F), 0, W2_bt)
    a2_bt = _edge_scatter(m2_bt, sb, db)
    a2_inc = _edge_scatter(m2_inc, si, di)
    out_group = _tc_final(a2_bt, deg_parts, 1, b2_bt.reshape(1, F))
    out_user = _tc_final(a2_inc, deg_parts, 3, b2_inc.reshape(1, F))
    return out_user, out_group
